# baseline (speedup 1.0000x reference)
import functools

import jax
import jax.numpy as jnp
from jax import lax
from jax.experimental import pallas as pl
from jax.experimental.pallas import tpu as pltpu
from jax.experimental.pallas import tpu_sc as plsc

VOCAB = 100000
D = 128
B = 16384
NPOS = 10
NNEG = 50

DW = D // 2

NC = 2
NS = 16
NW = NC * NS
CHUNK = B // NW
G = 8
NG = CHUNK // G
L = 16

_NEG_CHUNKS = ((0, 128), (128, 128), (256, 128), (384, 16))


def _log_sigmoid(x):
    t = jnp.exp(-jnp.abs(x))
    r = t / (t + 2.0)
    r2 = r * r
    p = 1.0 + r2 * (1.0 / 3.0 + r2 * (1.0 / 5.0 + r2 * (1.0 / 7.0 + r2 * (1.0 / 9.0))))
    return jnp.minimum(x, 0.0) - 2.0 * r * p


def _pack_table(tab):
    w = jax.lax.bitcast_convert_type(tab, jnp.uint32)
    lo, hi = w[:, :DW], w[:, DW:]

    def rne16(x):
        return (x + 0x7FFF + ((x >> 16) & 1)) >> 16

    packed = rne16(lo) | (rne16(hi) << 16)
    return jax.lax.bitcast_convert_type(packed, jnp.int32)


def _body(in_l, pos_l, neg_l, in_tab, out_tab, out,
          idx_in0, idx_p0, idx_n0, Ri0, Rall0,
          idx_in1, idx_p1, idx_n1, Ri1, Rall1, outbuf, semA, semB):
    wid = lax.axis_index("s") * NC + lax.axis_index("c")
    bufs = ((idx_in0, idx_p0, idx_n0, Ri0, Rall0, semA),
            (idx_in1, idx_p1, idx_n1, Ri1, Rall1, semB))
    lanes = lax.iota(jnp.int32, L)
    sign0 = jnp.where(lanes < NPOS, 1.0, -1.0)
    mask0 = lanes < NPOS + 2

    def copies(g, p, issue):
        base = wid * CHUNK + g * G
        idx_in, idx_p, idx_n, Ri, Rall, sem = bufs[p]
        if issue:
            pltpu.sync_copy(in_l.at[pl.ds(base, G)], idx_in)
            pltpu.sync_copy(pos_l.at[pl.ds(base * NPOS, G * NPOS)], idx_p)
            pltpu.sync_copy(neg_l.at[pl.ds(base * NNEG, G * NNEG)], idx_n)
        mk = pltpu.async_copy if issue else pltpu.make_async_copy
        descs = [
            mk(in_tab.at[idx_in], Ri, sem),
            mk(out_tab.at[idx_p], Rall.at[pl.ds(0, G * NPOS)], sem),
        ]
        for off, ln in _NEG_CHUNKS:
            descs.append(
                mk(out_tab.at[idx_n.at[pl.ds(off, ln)]],
                   Rall.at[pl.ds(G * NPOS + off, ln)], sem))
        return descs

    def compute(g, p):
        Ri_p = bufs[p][3]
        Rall_p = bufs[p][4]

        def elem(e, loss_vec):
            nbase = G * NPOS + e * NNEG
            rows = [jnp.where(lanes < NPOS, e * NPOS + lanes,
                              jnp.minimum(nbase + 38 + lanes,
                                          G * (NPOS + NNEG) - 1))]
            rows += [nbase + k * L + lanes for k in range(3)]
            esplat = jnp.full((L,), e, jnp.int32)

            def unpack2(word):
                hi = plsc.bitcast(word, jnp.float32)
                lo = plsc.bitcast(lax.shift_left(word, 16), jnp.float32)
                return hi, lo

            def dstep(ww, accs):
                dcol = (lanes + ww) & (DW - 1)
                sv_hi, sv_lo = unpack2(plsc.load_gather(Ri_p, [esplat, dcol]))
                new = []
                for m in range(4):
                    hi, lo = unpack2(
                        plsc.load_gather(Rall_p, [rows[m], dcol]))
                    new.append(accs[m] + hi * sv_hi + lo * sv_lo)
                return tuple(new)

            zero = jnp.zeros((L,), jnp.float32)
            dots = lax.fori_loop(0, DW, dstep, (zero,) * 4, unroll=16)

            acc = jnp.where(mask0, _log_sigmoid(dots[0] * sign0), 0.0)
            for k in range(1, 4):
                acc = acc + _log_sigmoid(-dots[k])
            loss = jnp.full((L,), -jnp.sum(acc), jnp.float32)
            return jnp.where(lanes == e, loss, loss_vec)

        loss_vec = lax.fori_loop(0, G, elem, jnp.zeros((L,), jnp.float32))
        plsc.store_scatter(outbuf, [g * G + lanes], loss_vec,
                           mask=lanes < G)

    def do_group(g, p):
        for desc in copies(g, p, issue=False):
            desc.wait()
        compute(g, p)

        @pl.when(g + 2 < NG)
        def _prefetch():
            copies(g + 2, p, issue=True)

    copies(0, 0, issue=True)
    copies(1, 1, issue=True)

    def pair(i, _):
        do_group(2 * i, 0)
        do_group(2 * i + 1, 1)
        return 0

    lax.fori_loop(0, NG // 2, pair, 0)
    pltpu.sync_copy(outbuf, out.at[pl.ds(wid * CHUNK, CHUNK)])


@functools.cache
def _sc_call():
    return functools.partial(
        pl.kernel,
        out_type=jax.ShapeDtypeStruct((B,), jnp.float32),
        mesh=plsc.VectorSubcoreMesh(core_axis_name="c", subcore_axis_name="s",
                                    num_cores=NC, num_subcores=NS),
        compiler_params=pltpu.CompilerParams(needs_layout_passes=False,
                                             use_tc_tiling_on_sc=False),
        scratch_types=(
            [pltpu.VMEM((G,), jnp.int32),
             pltpu.VMEM((G * NPOS,), jnp.int32),
             pltpu.VMEM((G * NNEG,), jnp.int32),
             pltpu.VMEM((G, DW), jnp.int32),
             pltpu.VMEM((G * (NPOS + NNEG), DW), jnp.int32),
             ] * 2
            + [pltpu.VMEM((CHUNK,), jnp.float32),
               pltpu.SemaphoreType.DMA,
               pltpu.SemaphoreType.DMA]),
    )(_body)


def kernel(input_labels, positive_labels, negative_labels, input_table,
           output_table):
    pos_flat = positive_labels.reshape(-1)
    neg_flat = negative_labels.reshape(-1)
    return _sc_call()(input_labels, pos_flat, neg_flat,
                      _pack_table(input_table), _pack_table(output_table))


# speedup vs baseline: 1.4767x; 1.4767x over previous
import functools

import jax
import jax.numpy as jnp
from jax import lax
from jax.experimental import pallas as pl
from jax.experimental.pallas import tpu as pltpu
from jax.experimental.pallas import tpu_sc as plsc

VOCAB = 100000
D = 128
B = 16384
NPOS = 10
NNEG = 50

DW = D // 2

NC = 2
NS = 16
NW = NC * NS
CHUNK = B // NW
G = 8
NG = CHUNK // G
L = 16

_NEG_CHUNKS = ((0, 128), (128, 128), (256, 128), (384, 16))


def _log_sigmoid(x):
    t = jnp.exp(-jnp.abs(x))
    r = t / (t + 2.0)
    r2 = r * r
    p = 1.0 + r2 * (1.0 / 3.0 + r2 * (1.0 / 5.0 + r2 * (1.0 / 7.0 + r2 * (1.0 / 9.0))))
    return jnp.minimum(x, 0.0) - 2.0 * r * p


def _pack_table(tab):
    w = jax.lax.bitcast_convert_type(tab, jnp.uint32)
    lo, hi = w[:, :DW], w[:, DW:]

    def rne16(x):
        return (x + 0x7FFF + ((x >> 16) & 1)) >> 16

    packed = rne16(lo) | (rne16(hi) << 16)
    return jax.lax.bitcast_convert_type(packed, jnp.int32)


def _body(in_l, pos_l, neg_l, in_tab, out_tab, out,
          idx_in0, idx_p0, idx_n0, Ri0, Rall0,
          idx_in1, idx_p1, idx_n1, Ri1, Rall1, outbuf, semA, semB):
    wid = lax.axis_index("s") * NC + lax.axis_index("c")
    bufs = ((idx_in0, idx_p0, idx_n0, Ri0, Rall0, semA),
            (idx_in1, idx_p1, idx_n1, Ri1, Rall1, semB))
    lanes = lax.iota(jnp.int32, L)
    sign0 = jnp.where(lanes < NPOS, 1.0, -1.0)
    mask0 = lanes < NPOS + 2

    def copies(g, p, issue):
        base = wid * CHUNK + g * G
        idx_in, idx_p, idx_n, Ri, Rall, sem = bufs[p]
        if issue:
            pltpu.sync_copy(in_l.at[pl.ds(base, G)], idx_in)
            pltpu.sync_copy(pos_l.at[pl.ds(base * NPOS, G * NPOS)], idx_p)
            pltpu.sync_copy(neg_l.at[pl.ds(base * NNEG, G * NNEG)], idx_n)
        mk = pltpu.async_copy if issue else pltpu.make_async_copy
        descs = [
            mk(in_tab.at[idx_in], Ri, sem),
            mk(out_tab.at[idx_p], Rall.at[pl.ds(0, G * NPOS)], sem),
        ]
        for off, ln in _NEG_CHUNKS:
            descs.append(
                mk(out_tab.at[idx_n.at[pl.ds(off, ln)]],
                   Rall.at[pl.ds(G * NPOS + off, ln)], sem))
        return descs

    def compute(g, p):
        Ri_p = bufs[p][3]
        Rall_p = bufs[p][4]

        def elem(e, loss_vec):
            nbase = G * NPOS + e * NNEG
            rows = [jnp.where(lanes < NPOS, e * NPOS + lanes,
                              jnp.minimum(nbase + 38 + lanes,
                                          G * (NPOS + NNEG) - 1))]
            rows += [nbase + k * L + lanes for k in range(3)]
            esplat = jnp.full((L,), e, jnp.int32)

            def unpack2(word):
                hi = plsc.bitcast(word, jnp.float32)
                lo = plsc.bitcast(lax.shift_left(word, 16), jnp.float32)
                return hi, lo

            def dstep(ww, accs):
                dcol = (lanes + ww) & (DW - 1)
                sv_hi, sv_lo = unpack2(plsc.load_gather(Ri_p, [esplat, dcol]))
                new = []
                for m in range(4):
                    hi, lo = unpack2(
                        plsc.load_gather(Rall_p, [rows[m], dcol]))
                    new.append(accs[m] + hi * sv_hi + lo * sv_lo)
                return tuple(new)

            zero = jnp.zeros((L,), jnp.float32)
            dots = lax.fori_loop(0, DW, dstep, (zero,) * 4, unroll=4)

            acc = jnp.where(mask0, _log_sigmoid(dots[0] * sign0), 0.0)
            for k in range(1, 4):
                acc = acc + _log_sigmoid(-dots[k])
            loss = jnp.full((L,), -jnp.sum(acc), jnp.float32)
            return jnp.where(lanes == e, loss, loss_vec)

        loss_vec = lax.fori_loop(0, G, elem, jnp.zeros((L,), jnp.float32))
        plsc.store_scatter(outbuf, [g * G + lanes], loss_vec,
                           mask=lanes < G)

    def do_group(g, p):
        for desc in copies(g, p, issue=False):
            desc.wait()
        compute(g, p)

        @pl.when(g + 2 < NG)
        def _prefetch():
            copies(g + 2, p, issue=True)

    copies(0, 0, issue=True)
    copies(1, 1, issue=True)

    def pair(i, _):
        do_group(2 * i, 0)
        do_group(2 * i + 1, 1)
        return 0

    lax.fori_loop(0, NG // 2, pair, 0)
    pltpu.sync_copy(outbuf, out.at[pl.ds(wid * CHUNK, CHUNK)])


@functools.cache
def _sc_call():
    return functools.partial(
        pl.kernel,
        out_type=jax.ShapeDtypeStruct((B,), jnp.float32),
        mesh=plsc.VectorSubcoreMesh(core_axis_name="c", subcore_axis_name="s",
                                    num_cores=NC, num_subcores=NS),
        compiler_params=pltpu.CompilerParams(needs_layout_passes=False,
                                             use_tc_tiling_on_sc=False),
        scratch_types=(
            [pltpu.VMEM((G,), jnp.int32),
             pltpu.VMEM((G * NPOS,), jnp.int32),
             pltpu.VMEM((G * NNEG,), jnp.int32),
             pltpu.VMEM((G, DW), jnp.int32),
             pltpu.VMEM((G * (NPOS + NNEG), DW), jnp.int32),
             ] * 2
            + [pltpu.VMEM((CHUNK,), jnp.float32),
               pltpu.SemaphoreType.DMA,
               pltpu.SemaphoreType.DMA]),
    )(_body)


def kernel(input_labels, positive_labels, negative_labels, input_table,
           output_table):
    pos_flat = positive_labels.reshape(-1)
    neg_flat = negative_labels.reshape(-1)
    return _sc_call()(input_labels, pos_flat, neg_flat,
                      _pack_table(input_table), _pack_table(output_table))


# speedup vs baseline: 2.1500x; 1.4559x over previous
import functools

import jax
import jax.numpy as jnp
from jax import lax
from jax.experimental import pallas as pl
from jax.experimental.pallas import tpu as pltpu
from jax.experimental.pallas import tpu_sc as plsc

VOCAB = 100000
D = 128
B = 16384
NPOS = 10
NNEG = 50

DW = D // 2

NC = 2
NS = 16
NW = NC * NS
CHUNK = B // NW
G = 8
NG = CHUNK // G
NR = NPOS + NNEG
L = 16

_NEG_CHUNKS = ((0, 128), (128, 128), (256, 128), (384, 16))


def _log_sigmoid(x):
    t = jnp.exp(-jnp.abs(x))
    r = t / (t + 2.0)
    r2 = r * r
    p = 1.0 + r2 * (1.0 / 3.0 + r2 * (1.0 / 5.0 + r2 * (1.0 / 7.0 + r2 * (1.0 / 9.0))))
    return jnp.minimum(x, 0.0) - 2.0 * r * p


def _body(in_l, pos_l, neg_l, in_tab, out_tab, out,
          idx_in0, idx_p0, idx_n0, Ri0, Rall0,
          idx_in1, idx_p1, idx_n1, Ri1, Rall1,
          outbuf, semA, semB, semL):
    wid = lax.axis_index("s") * NC + lax.axis_index("c")
    bufs = ((idx_in0, idx_p0, idx_n0, Ri0, Rall0, semA),
            (idx_in1, idx_p1, idx_n1, Ri1, Rall1, semB))
    lanes = lax.iota(jnp.int32, L)
    sign0 = jnp.where(lanes < NPOS, 1.0, -1.0)
    mask0 = lanes < NPOS + 2

    def label_copies(g, p, mk):
        base = wid * CHUNK + g * G
        idx_in, idx_p, idx_n = bufs[p][:3]
        return [
            mk(in_l.at[pl.ds(base, G)], idx_in, semL),
            mk(pos_l.at[pl.ds(base * NPOS, G * NPOS)], idx_p, semL),
            mk(neg_l.at[pl.ds(base * NNEG, G * NNEG)], idx_n, semL),
        ]

    def row_copies(p, mk):
        idx_in, idx_p, idx_n, Ri, Rall, sem = bufs[p]
        descs = [
            mk(in_tab.at[idx_in], Ri, sem),
            mk(out_tab.at[idx_p], Rall.at[pl.ds(0, G * NPOS)], sem),
        ]
        for off, ln in _NEG_CHUNKS:
            descs.append(
                mk(out_tab.at[idx_n.at[pl.ds(off, ln)]],
                   Rall.at[pl.ds(G * NPOS + off, ln)], sem))
        return descs

    def compute(g, p):
        Ri_p = bufs[p][3]
        Rall_p = bufs[p][4]

        def elem(e, loss_vec):
            nbase = G * NPOS + e * NNEG
            rows = [jnp.where(lanes < NPOS, e * NPOS + lanes,
                              jnp.minimum(nbase + 38 + lanes, G * NR - 1))]
            rows += [nbase + k * L + lanes for k in range(3)]
            esplat = jnp.full((L,), e, jnp.int32)

            def dstep(dd, accs):
                dcol = (lanes + dd) & (D - 1)
                sv = plsc.load_gather(Ri_p, [esplat, dcol])
                return tuple(
                    accs[m] + plsc.load_gather(Rall_p, [rows[m], dcol]) * sv
                    for m in range(4))

            zero = jnp.zeros((L,), jnp.float32)
            dots = lax.fori_loop(0, D, dstep, (zero,) * 4, unroll=16)

            acc = jnp.where(mask0, _log_sigmoid(dots[0] * sign0), 0.0)
            for k in range(1, 4):
                acc = acc + _log_sigmoid(-dots[k])
            loss = jnp.full((L,), -jnp.sum(acc), jnp.float32)
            return jnp.where(lanes == e, loss, loss_vec)

        loss_vec = lax.fori_loop(0, G, elem, jnp.zeros((L,), jnp.float32))
        plsc.store_scatter(outbuf, [g * G + lanes], loss_vec,
                           mask=lanes < G)

    def do_group(g, p):
        for desc in row_copies(p, pltpu.make_async_copy):
            desc.wait()

        @pl.when(g + 2 < NG)
        def _labels():
            label_copies(g + 2, p, pltpu.async_copy)

        compute(g, p)

        @pl.when(g + 2 < NG)
        def _streams():
            for desc in label_copies(g + 2, p, pltpu.make_async_copy):
                desc.wait()
            row_copies(p, pltpu.async_copy)

    for pp in (0, 1):
        for desc in label_copies(pp, pp, pltpu.async_copy):
            desc.wait()
        row_copies(pp, pltpu.async_copy)

    def pair(i, _):
        do_group(2 * i, 0)
        do_group(2 * i + 1, 1)
        return 0

    lax.fori_loop(0, NG // 2, pair, 0)
    pltpu.sync_copy(outbuf, out.at[pl.ds(wid * CHUNK, CHUNK)])


@functools.cache
def _sc_call():
    return functools.partial(
        pl.kernel,
        out_type=jax.ShapeDtypeStruct((B,), jnp.float32),
        mesh=plsc.VectorSubcoreMesh(core_axis_name="c", subcore_axis_name="s",
                                    num_cores=NC, num_subcores=NS),
        compiler_params=pltpu.CompilerParams(needs_layout_passes=False,
                                             use_tc_tiling_on_sc=False),
        scratch_types=(
            [pltpu.VMEM((G,), jnp.int32),
             pltpu.VMEM((G * NPOS,), jnp.int32),
             pltpu.VMEM((G * NNEG,), jnp.int32),
             pltpu.VMEM((G, D), jnp.float32),
             pltpu.VMEM((G * NR, D), jnp.float32),
             ] * 2
            + [pltpu.VMEM((CHUNK,), jnp.float32),
               pltpu.SemaphoreType.DMA,
               pltpu.SemaphoreType.DMA,
               pltpu.SemaphoreType.DMA]),
    )(_body)


def kernel(input_labels, positive_labels, negative_labels, input_table,
           output_table):
    pos_flat = positive_labels.reshape(-1)
    neg_flat = negative_labels.reshape(-1)
    return _sc_call()(input_labels, pos_flat, neg_flat, input_table,
                      output_table)


# speedup vs baseline: 2.1556x; 1.0026x over previous
import functools

import jax
import jax.numpy as jnp
from jax import lax
from jax.experimental import pallas as pl
from jax.experimental.pallas import tpu as pltpu
from jax.experimental.pallas import tpu_sc as plsc

VOCAB = 100000
D = 128
B = 16384
NPOS = 10
NNEG = 50

DW = D // 2

NC = 2
NS = 16
NW = NC * NS
CHUNK = B // NW
G = 8
NG = CHUNK // G
NR = NPOS + NNEG
L = 16

_NEG_CHUNKS = ((0, 400),)


def _log_sigmoid(x):
    t = jnp.exp(-jnp.abs(x))
    r = t / (t + 2.0)
    r2 = r * r
    p = 1.0 + r2 * (1.0 / 3.0 + r2 * (1.0 / 5.0 + r2 * (1.0 / 7.0 + r2 * (1.0 / 9.0))))
    return jnp.minimum(x, 0.0) - 2.0 * r * p


def _body(in_l, pos_l, neg_l, in_tab, out_tab, out,
          idx_in0, idx_p0, idx_n0, Ri0, Rall0,
          idx_in1, idx_p1, idx_n1, Ri1, Rall1,
          outbuf, semA, semB, semL):
    wid = lax.axis_index("s") * NC + lax.axis_index("c")
    bufs = ((idx_in0, idx_p0, idx_n0, Ri0, Rall0, semA),
            (idx_in1, idx_p1, idx_n1, Ri1, Rall1, semB))
    lanes = lax.iota(jnp.int32, L)
    sign0 = jnp.where(lanes < NPOS, 1.0, -1.0)
    mask0 = lanes < NPOS + 2

    def label_copies(g, p, mk):
        base = wid * CHUNK + g * G
        idx_in, idx_p, idx_n = bufs[p][:3]
        return [
            mk(in_l.at[pl.ds(base, G)], idx_in, semL),
            mk(pos_l.at[pl.ds(base * NPOS, G * NPOS)], idx_p, semL),
            mk(neg_l.at[pl.ds(base * NNEG, G * NNEG)], idx_n, semL),
        ]

    def row_copies(p, mk):
        idx_in, idx_p, idx_n, Ri, Rall, sem = bufs[p]
        descs = [
            mk(in_tab.at[idx_in], Ri, sem),
            mk(out_tab.at[idx_p], Rall.at[pl.ds(0, G * NPOS)], sem),
        ]
        for off, ln in _NEG_CHUNKS:
            descs.append(
                mk(out_tab.at[idx_n.at[pl.ds(off, ln)]],
                   Rall.at[pl.ds(G * NPOS + off, ln)], sem))
        return descs

    def compute(g, p):
        Ri_p = bufs[p][3]
        Rall_p = bufs[p][4]

        def elem(e, loss_vec):
            nbase = G * NPOS + e * NNEG
            rows = [jnp.where(lanes < NPOS, e * NPOS + lanes,
                              jnp.minimum(nbase + 38 + lanes, G * NR - 1))]
            rows += [nbase + k * L + lanes for k in range(3)]
            esplat = jnp.full((L,), e, jnp.int32)

            def dstep(dd, accs):
                dcol = (lanes + dd) & (D - 1)
                sv = plsc.load_gather(Ri_p, [esplat, dcol])
                return tuple(
                    accs[m] + plsc.load_gather(Rall_p, [rows[m], dcol]) * sv
                    for m in range(4))

            zero = jnp.zeros((L,), jnp.float32)
            dots = lax.fori_loop(0, D, dstep, (zero,) * 4, unroll=16)

            acc = jnp.where(mask0, _log_sigmoid(dots[0] * sign0), 0.0)
            for k in range(1, 4):
                acc = acc + _log_sigmoid(-dots[k])
            loss = jnp.full((L,), -jnp.sum(acc), jnp.float32)
            return jnp.where(lanes == e, loss, loss_vec)

        loss_vec = lax.fori_loop(0, G, elem, jnp.zeros((L,), jnp.float32))
        plsc.store_scatter(outbuf, [g * G + lanes], loss_vec,
                           mask=lanes < G)

    def do_group(g, p):
        for desc in row_copies(p, pltpu.make_async_copy):
            desc.wait()

        @pl.when(g + 2 < NG)
        def _labels():
            label_copies(g + 2, p, pltpu.async_copy)

        compute(g, p)

        @pl.when(g + 2 < NG)
        def _streams():
            for desc in label_copies(g + 2, p, pltpu.make_async_copy):
                desc.wait()
            row_copies(p, pltpu.async_copy)

    for pp in (0, 1):
        for desc in label_copies(pp, pp, pltpu.async_copy):
            desc.wait()
        row_copies(pp, pltpu.async_copy)

    def pair(i, _):
        do_group(2 * i, 0)
        do_group(2 * i + 1, 1)
        return 0

    lax.fori_loop(0, NG // 2, pair, 0)
    pltpu.sync_copy(outbuf, out.at[pl.ds(wid * CHUNK, CHUNK)])


@functools.cache
def _sc_call():
    return functools.partial(
        pl.kernel,
        out_type=jax.ShapeDtypeStruct((B,), jnp.float32),
        mesh=plsc.VectorSubcoreMesh(core_axis_name="c", subcore_axis_name="s",
                                    num_cores=NC, num_subcores=NS),
        compiler_params=pltpu.CompilerParams(needs_layout_passes=False,
                                             use_tc_tiling_on_sc=False),
        scratch_types=(
            [pltpu.VMEM((G,), jnp.int32),
             pltpu.VMEM((G * NPOS,), jnp.int32),
             pltpu.VMEM((G * NNEG,), jnp.int32),
             pltpu.VMEM((G, D), jnp.float32),
             pltpu.VMEM((G * NR, D), jnp.float32),
             ] * 2
            + [pltpu.VMEM((CHUNK,), jnp.float32),
               pltpu.SemaphoreType.DMA,
               pltpu.SemaphoreType.DMA,
               pltpu.SemaphoreType.DMA]),
    )(_body)


def kernel(input_labels, positive_labels, negative_labels, input_table,
           output_table):
    pos_flat = positive_labels.reshape(-1)
    neg_flat = negative_labels.reshape(-1)
    return _sc_call()(input_labels, pos_flat, neg_flat, input_table,
                      output_table)
